# Initial kernel scaffold; baseline (speedup 1.0000x reference)
import jax
import jax.numpy as jnp
from jax.experimental import pallas as pl


def kernel(edge_index, W1, al1, ar1, b1, W2, al2, ar2, b2, Wc, bc):
    raise NotImplementedError("write your pallas kernel here")


# speedup vs baseline: 357.6611x; 357.6611x over previous
import functools

import jax
import jax.numpy as jnp
from jax import lax
from jax.experimental import pallas as pl
from jax.experimental.pallas import tpu as pltpu
from jax.experimental.pallas import tpu_sc as plsc

N = 50000
E = 800000
H = 2
D = 32
NC = 10

NPAD = 51200
NR, NL = 400, 128
DUMMY = N
NWORK = 32
EP = 819200
EPW = EP // NWORK
CH = 3200
NCHUNK = EPW // CH
TSL = NPAD // 16

_mesh = plsc.VectorSubcoreMesh(core_axis_name="c", subcore_axis_name="s")

_i32 = jnp.int32
_f32 = jnp.float32


@functools.partial(
    pl.kernel,
    out_type=jax.ShapeDtypeStruct((2, NPAD), _f32),
    mesh=_mesh,
    compiler_params=pltpu.CompilerParams(needs_layout_passes=False, use_tc_tiling_on_sc=False),
    scratch_types=[
        pltpu.VMEM((CH,), _i32),
        pltpu.VMEM((CH,), _f32),
        pltpu.VMEM_SHARED((NPAD,), _f32),
    ],
)
def _sc_deg(dst_hbm, ones_hbm, zplane_hbm, out_hbm, didx, onesv, acc):
    c = lax.axis_index("c")
    s = lax.axis_index("s")
    wid = s * 2 + c
    pltpu.sync_copy(zplane_hbm.at[pl.ds(s * TSL, TSL)], acc.at[pl.ds(s * TSL, TSL)])
    pltpu.sync_copy(ones_hbm, onesv)
    plsc.subcore_barrier()
    ebase = wid * EPW
    for k in range(NCHUNK):
        pltpu.sync_copy(dst_hbm.at[pl.ds(ebase + k * CH, CH)], didx)
        pltpu.sync_copy(onesv, acc.at[didx], add=True)
    plsc.subcore_barrier()
    pltpu.sync_copy(acc.at[pl.ds(s * TSL, TSL)], out_hbm.at[c, pl.ds(s * TSL, TSL)])


@functools.partial(
    pl.kernel,
    out_type=jax.ShapeDtypeStruct((2, 4, NPAD), _f32),
    mesh=_mesh,
    compiler_params=pltpu.CompilerParams(needs_layout_passes=False, use_tc_tiling_on_sc=False),
    scratch_types=[
        pltpu.VMEM((NPAD,), _f32),
        pltpu.VMEM((CH,), _i32),
        pltpu.VMEM((CH,), _i32),
        pltpu.VMEM((CH,), _f32),
        pltpu.VMEM((CH,), _f32),
        pltpu.VMEM((CH,), _f32),
        pltpu.VMEM((CH,), _f32),
        pltpu.VMEM((6, 16), _f32),
        pltpu.VMEM_SHARED((NPAD,), _f32),
        pltpu.VMEM_SHARED((NPAD,), _f32),
        pltpu.VMEM_SHARED((NPAD,), _f32),
        pltpu.VMEM_SHARED((NPAD,), _f32),
    ],
)
def _sc_l1(src_hbm, dst_hbm, deg_hbm, c1_hbm, zplane_hbm, out_hbm,
           degrep, sidx, didx, ex0, ex1, t0, t1, cbuf,
           acc0, acc1, acc2, acc3):
    c = lax.axis_index("c")
    s = lax.axis_index("s")
    wid = s * 2 + c
    zsl = pl.ds(s * TSL, TSL)
    for acc in (acc0, acc1, acc2, acc3):
        pltpu.sync_copy(zplane_hbm.at[zsl], acc.at[zsl])
    pltpu.sync_copy(deg_hbm, degrep)
    pltpu.sync_copy(c1_hbm, cbuf)
    plsc.subcore_barrier()
    a0 = cbuf[0]
    a1 = cbuf[1]
    c0 = cbuf[2]
    c1v = cbuf[3]
    m0 = cbuf[4]
    m1 = cbuf[5]
    ebase = wid * EPW
    for k in range(NCHUNK):
        pltpu.sync_copy(src_hbm.at[pl.ds(ebase + k * CH, CH)], sidx)
        pltpu.sync_copy(dst_hbm.at[pl.ds(ebase + k * CH, CH)], didx)

        def body(i, _):
            sl = pl.ds(i * 16, 16)
            sv = sidx[sl]
            dv = didx[sl]
            dsv = plsc.load_gather(degrep, [sv])
            ddv = plsc.load_gather(degrep, [dv])
            ta = a0 * dsv + c0 * ddv
            ea = jnp.exp(jnp.maximum(ta, 0.2 * ta) - m0)
            ex0[sl] = ea
            t0[sl] = dsv * ea
            tb = a1 * dsv + c1v * ddv
            eb = jnp.exp(jnp.maximum(tb, 0.2 * tb) - m1)
            ex1[sl] = eb
            t1[sl] = dsv * eb
            return 0

        lax.fori_loop(0, CH // 16, body, 0)
        pltpu.sync_copy(ex0, acc0.at[didx], add=True)
        pltpu.sync_copy(ex1, acc1.at[didx], add=True)
        pltpu.sync_copy(t0, acc2.at[didx], add=True)
        pltpu.sync_copy(t1, acc3.at[didx], add=True)
    plsc.subcore_barrier()
    for j, acc in enumerate((acc0, acc1, acc2, acc3)):
        pltpu.sync_copy(acc.at[zsl], out_hbm.at[c, j, zsl])


@functools.partial(
    pl.kernel,
    out_type=(jax.ShapeDtypeStruct((2, 2, NPAD), _f32),
              jax.ShapeDtypeStruct((2, EP), _f32)),
    mesh=_mesh,
    compiler_params=pltpu.CompilerParams(needs_layout_passes=False, use_tc_tiling_on_sc=False),
    scratch_types=[
        pltpu.VMEM((NPAD,), _f32),
        pltpu.VMEM((NPAD,), _f32),
        pltpu.VMEM((CH,), _i32),
        pltpu.VMEM((CH,), _i32),
        pltpu.VMEM((CH,), _f32),
        pltpu.VMEM((2, 16), _f32),
        pltpu.VMEM_SHARED((NPAD,), _f32),
        pltpu.VMEM_SHARED((NPAD,), _f32),
    ],
)
def _sc_l2a(src_hbm, dst_hbm, elcat_hbm, ercat_hbm, c2_hbm, zplane_hbm,
            s2_hbm, ex2_hbm, elrep, errep, sidx, didx, exv, cbuf,
            acc0, acc1):
    c = lax.axis_index("c")
    s = lax.axis_index("s")
    wid = s * 2 + c
    zsl = pl.ds(s * TSL, TSL)
    pltpu.sync_copy(zplane_hbm.at[zsl], acc0.at[zsl])
    pltpu.sync_copy(zplane_hbm.at[zsl], acc1.at[zsl])
    pltpu.sync_copy(c2_hbm, cbuf)
    plsc.subcore_barrier()
    ebase = wid * EPW
    for h, acc in ((0, acc0), (1, acc1)):
        pltpu.sync_copy(elcat_hbm.at[pl.ds(h * NPAD, NPAD)], elrep)
        pltpu.sync_copy(ercat_hbm.at[pl.ds(h * NPAD, NPAD)], errep)
        mh = cbuf[h]
        for k in range(NCHUNK):
            pltpu.sync_copy(src_hbm.at[pl.ds(ebase + k * CH, CH)], sidx)
            pltpu.sync_copy(dst_hbm.at[pl.ds(ebase + k * CH, CH)], didx)

            def body(i, _):
                sl = pl.ds(i * 16, 16)
                sv = sidx[sl]
                dv = didx[sl]
                el = plsc.load_gather(elrep, [sv])
                er = plsc.load_gather(errep, [dv])
                ta = el + er
                exv[sl] = jnp.exp(jnp.maximum(ta, 0.2 * ta) - mh)
                return 0

            lax.fori_loop(0, CH // 16, body, 0)
            pltpu.sync_copy(exv, acc.at[didx], add=True)
            pltpu.sync_copy(exv, ex2_hbm.at[h, pl.ds(ebase + k * CH, CH)])
    plsc.subcore_barrier()
    pltpu.sync_copy(acc0.at[zsl], s2_hbm.at[c, 0, zsl])
    pltpu.sync_copy(acc1.at[zsl], s2_hbm.at[c, 1, zsl])


@functools.partial(
    pl.kernel,
    out_type=jax.ShapeDtypeStruct((2, 2, NPAD), _f32),
    mesh=_mesh,
    compiler_params=pltpu.CompilerParams(needs_layout_passes=False, use_tc_tiling_on_sc=False),
    scratch_types=[
        pltpu.VMEM((NPAD,), _f32),
        pltpu.VMEM((TSL,), _f32),
        pltpu.VMEM((TSL,), _f32),
        pltpu.VMEM((CH,), _i32),
        pltpu.VMEM((CH,), _i32),
        pltpu.VMEM((CH,), _f32),
        pltpu.VMEM((CH,), _f32),
        pltpu.VMEM_SHARED((2 * NPAD,), _f32),
        pltpu.VMEM_SHARED((NPAD,), _f32),
        pltpu.VMEM_SHARED((NPAD,), _f32),
    ],
)
def _sc_l2b(src_hbm, dst_hbm, ex2_hbm, s2p_hbm, zplane_hbm, out_hbm,
            s2rep, stga, stgb, sidx, didx, exv, gv,
            ssum, accU0, accU1):
    c = lax.axis_index("c")
    s = lax.axis_index("s")
    wid = s * 2 + c
    zsl = pl.ds(s * TSL, TSL)
    pltpu.sync_copy(zplane_hbm.at[zsl], accU0.at[zsl])
    pltpu.sync_copy(zplane_hbm.at[zsl], accU1.at[zsl])
    for p in range(2):
        pltpu.sync_copy(s2p_hbm.at[0, p, zsl], stga)
        pltpu.sync_copy(s2p_hbm.at[1, p, zsl], stgb)

        def addb(i, _):
            sl = pl.ds(i * 16, 16)
            stga[sl] = stga[sl] + stgb[sl]
            return 0

        lax.fori_loop(0, TSL // 16, addb, 0)
        pltpu.sync_copy(stga, ssum.at[pl.ds(p * NPAD + s * TSL, TSL)])
    plsc.subcore_barrier()
    ebase = wid * EPW
    for h, acc in ((0, accU0), (1, accU1)):
        pltpu.sync_copy(ssum.at[pl.ds(h * NPAD, NPAD)], s2rep)
        for k in range(NCHUNK):
            pltpu.sync_copy(src_hbm.at[pl.ds(ebase + k * CH, CH)], sidx)
            pltpu.sync_copy(dst_hbm.at[pl.ds(ebase + k * CH, CH)], didx)
            pltpu.sync_copy(ex2_hbm.at[h, pl.ds(ebase + k * CH, CH)], exv)

            def body(i, _):
                sl = pl.ds(i * 16, 16)
                dv = didx[sl]
                sh = plsc.load_gather(s2rep, [dv])
                gv[sl] = exv[sl] / jnp.maximum(sh, 1e-30)
                return 0

            lax.fori_loop(0, CH // 16, body, 0)
            pltpu.sync_copy(gv, acc.at[sidx], add=True)
    plsc.subcore_barrier()
    pltpu.sync_copy(accU0.at[zsl], out_hbm.at[c, 0, zsl])
    pltpu.sync_copy(accU1.at[zsl], out_hbm.at[c, 1, zsl])


def _valid_mask():
    r = lax.broadcasted_iota(_i32, (NR, NL), 0)
    l = lax.broadcasted_iota(_i32, (NR, NL), 1)
    return (r * NL + l) < N


def _tc_a_body(degp_ref, w1_ref, al1_ref, ar1_ref, deg_ref, c1_ref):
    dp = degp_ref[...]
    deg = dp[0] + dp[1]
    deg = jnp.where(_valid_mask(), deg, 0.0)
    deg_ref[...] = deg
    dmax = jnp.max(deg)
    w1a = w1_ref[:, 0:D]
    w1b = w1_ref[:, D:2 * D]
    a0 = jnp.sum(w1a * al1_ref[0:1, :])
    a1 = jnp.sum(w1b * al1_ref[1:2, :])
    c0 = jnp.sum(w1a * ar1_ref[0:1, :])
    c1 = jnp.sum(w1b * ar1_ref[1:2, :])
    bm0 = jnp.maximum(a0, 0.0) * dmax + jnp.maximum(c0, 0.0) * dmax
    bm1 = jnp.maximum(a1, 0.0) * dmax + jnp.maximum(c1, 0.0) * dmax
    m0 = jnp.maximum(bm0, 0.2 * bm0)
    m1 = jnp.maximum(bm1, 0.2 * bm1)
    li = lax.broadcasted_iota(_i32, (1, NL), 1)
    vals = jnp.zeros((1, NL), _f32)
    for idx, v in enumerate((a0, a1, c0, c1, m0, m1)):
        vals = jnp.where(li == idx, v, vals)
    c1_ref[...] = vals


def _coeffs(w1_ref, b1_ref, w2_ref):
    w1a = w1_ref[:, 0:D]
    w1b = w1_ref[:, D:2 * D]
    A0 = jnp.dot(0.5 * w1a, w2_ref[...], preferred_element_type=_f32)
    A1 = jnp.dot(0.5 * w1b, w2_ref[...], preferred_element_type=_f32)
    bb = 0.5 * (b1_ref[0:1, :] + b1_ref[1:2, :])
    bbW = jnp.dot(bb, w2_ref[...], preferred_element_type=_f32)
    return A0, A1, bbW


def _tc_b_body(accp_ref, w1_ref, b1_ref, w2_ref, al2_ref, ar2_ref,
               el_ref, er_ref, s_ref, c2_ref):
    P = accp_ref[...]
    Asum = P[0] + P[1]
    valid = _valid_mask()
    S0 = Asum[2] / jnp.maximum(Asum[0], 1e-30)
    S1 = Asum[3] / jnp.maximum(Asum[1], 1e-30)
    S0 = jnp.where(valid, S0, 0.0)
    S1 = jnp.where(valid, S1, 0.0)
    s_ref[0, :, :] = S0
    s_ref[1, :, :] = S1
    A0, A1, bbW = _coeffs(w1_ref, b1_ref, w2_ref)
    neg = jnp.float32(-1e30)
    mx = []
    for h in range(H):
        sl = slice(h * D, (h + 1) * D)
        pel0 = jnp.sum(A0[:, sl] * al2_ref[h:h + 1, :])
        pel1 = jnp.sum(A1[:, sl] * al2_ref[h:h + 1, :])
        qel = jnp.sum(bbW[:, sl] * al2_ref[h:h + 1, :])
        per0 = jnp.sum(A0[:, sl] * ar2_ref[h:h + 1, :])
        per1 = jnp.sum(A1[:, sl] * ar2_ref[h:h + 1, :])
        qer = jnp.sum(bbW[:, sl] * ar2_ref[h:h + 1, :])
        elh = pel0 * S0 + pel1 * S1 + qel
        erh = per0 * S0 + per1 * S1 + qer
        el_ref[h, :, :] = jnp.where(valid, elh, 0.0)
        er_ref[h, :, :] = jnp.where(valid, erh, 0.0)
        bm = (jnp.max(jnp.where(valid, elh, neg))
              + jnp.max(jnp.where(valid, erh, neg)))
        mx.append(jnp.maximum(bm, 0.2 * bm))
    li = lax.broadcasted_iota(_i32, (1, NL), 1)
    vals = jnp.zeros((1, NL), _f32)
    for idx, v in enumerate(mx):
        vals = jnp.where(li == idx, v, vals)
    c2_ref[...] = vals


def _tc_d_body(up_ref, s_ref, w1_ref, b1_ref, w2_ref, b2_ref, wc_ref, bc_ref,
               out_ref):
    UP = up_ref[...]
    valid = _valid_mask()
    U0 = jnp.where(valid, UP[0, 0] + UP[1, 0], 0.0)
    U1 = jnp.where(valid, UP[0, 1] + UP[1, 1], 0.0)
    S0 = s_ref[0]
    S1 = s_ref[1]
    A0, A1, bbW = _coeffs(w1_ref, b1_ref, w2_ref)
    hg = jnp.zeros((1, D), _f32)
    for h, Uh in enumerate((U0, U1)):
        sl = slice(h * D, (h + 1) * D)
        r0 = jnp.sum(Uh * S0)
        r1 = jnp.sum(Uh * S1)
        wt = jnp.sum(Uh)
        hg = hg + A0[:, sl] * r0 + A1[:, sl] * r1 + bbW[:, sl] * wt
    hg = hg / jnp.float32(N * H) + 0.5 * (b2_ref[0:1, :] + b2_ref[1:2, :])
    out_ref[...] = jnp.dot(hg, wc_ref[...], preferred_element_type=_f32) + bc_ref[...]


_tc_a = pl.pallas_call(
    _tc_a_body,
    out_shape=(jax.ShapeDtypeStruct((NR, NL), _f32),
               jax.ShapeDtypeStruct((1, NL), _f32)),
)

_tc_b = pl.pallas_call(
    _tc_b_body,
    out_shape=(jax.ShapeDtypeStruct((H, NR, NL), _f32),
               jax.ShapeDtypeStruct((H, NR, NL), _f32),
               jax.ShapeDtypeStruct((H, NR, NL), _f32),
               jax.ShapeDtypeStruct((1, NL), _f32)),
)

_tc_d = pl.pallas_call(
    _tc_d_body,
    out_shape=jax.ShapeDtypeStruct((1, NC), _f32),
)


def kernel(edge_index, W1, al1, ar1, b1, W2, al2, ar2, b2, Wc, bc):
    src = edge_index[0]
    dst = edge_index[1]
    fill = jnp.full((EP - E,), DUMMY, _i32)
    srcp = jnp.concatenate([src.astype(_i32), fill])
    dstp = jnp.concatenate([dst.astype(_i32), fill])
    zplane = jnp.zeros((NPAD,), _f32)
    ones_ch = jnp.ones((CH,), _f32)

    degp = _sc_deg(dstp, ones_ch, zplane)
    deg2d, c1 = _tc_a(degp.reshape(2, NR, NL), W1, al1, ar1)
    c1b = jnp.tile(c1[0, 0:6][:, None], (1, 16))
    acc1p = _sc_l1(srcp, dstp, deg2d.reshape(NPAD), c1b, zplane)
    el, er, Spl, c2 = _tc_b(acc1p.reshape(2, 4, NR, NL), W1, b1, W2, al2, ar2)
    elcat = el.reshape(2 * NPAD)
    ercat = er.reshape(2 * NPAD)
    c2b = jnp.tile(c2[0, 0:2][:, None], (1, 16))
    s2p, ex2 = _sc_l2a(srcp, dstp, elcat, ercat, c2b, zplane)
    up = _sc_l2b(srcp, dstp, ex2, s2p, zplane)
    out = _tc_d(up.reshape(2, 2, NR, NL), Spl, W1, b1, W2, b2, Wc,
                bc.reshape(1, NC))
    return out
